# baseline (speedup 1.0000x reference)
import functools

import jax
import jax.numpy as jnp
from jax import lax
from jax.experimental import pallas as pl
from jax.experimental.pallas import tpu as pltpu
from jax.experimental.pallas import tpu_sc as plsc

TILE_T = 256
TILE_E = 2048


def _distance_argmin_kernel(z_ref, et_ref, xn_ref, en_ref, idx_ref, loss_ref,
                            best_ref, bidx_ref):
    e = pl.program_id(1)
    n_e = pl.num_programs(1)
    z = z_ref[...]
    et = et_ref[...]
    xn = xn_ref[...]
    en = en_ref[...]
    mm = lax.dot_general(z.astype(jnp.bfloat16), et, (((1,), (0,)), ((), ())),
                         preferred_element_type=jnp.float32)
    dist = xn - 2.0 * mm + en
    lmin = jnp.min(dist, axis=1, keepdims=True)
    ii = lax.broadcasted_iota(jnp.int32, dist.shape, 1)
    lidx = (jnp.min(jnp.where(dist == lmin, ii, dist.shape[1]),
                    axis=1, keepdims=True) + e * TILE_E)

    @pl.when(e == 0)
    def _():
        best_ref[...] = lmin
        bidx_ref[...] = lidx

    @pl.when(e > 0)
    def _():
        better = lmin < best_ref[...]
        best_ref[...] = jnp.where(better, lmin, best_ref[...])
        bidx_ref[...] = jnp.where(better, lidx, bidx_ref[...])

    @pl.when(e == n_e - 1)
    def _():
        idx_ref[...] = bidx_ref[...]
        part = jnp.sum(best_ref[...], axis=0, keepdims=True)
        t = pl.program_id(0)

        @pl.when(t == 0)
        def _():
            loss_ref[...] = part

        @pl.when(t > 0)
        def _():
            loss_ref[...] = loss_ref[...] + part


def _distance_argmin(z_flat, emb_t, xn, en):
    bl, d = z_flat.shape
    ne = emb_t.shape[1]
    grid = (bl // TILE_T, ne // TILE_E)
    return pl.pallas_call(
        _distance_argmin_kernel,
        grid=grid,
        in_specs=[
            pl.BlockSpec((TILE_T, d), lambda t, e: (t, 0)),
            pl.BlockSpec((d, TILE_E), lambda t, e: (0, e)),
            pl.BlockSpec((TILE_T, 1), lambda t, e: (t, 0)),
            pl.BlockSpec((1, TILE_E), lambda t, e: (0, e)),
        ],
        out_specs=[
            pl.BlockSpec((TILE_T, 1), lambda t, e: (t, 0)),
            pl.BlockSpec((1, 1), lambda t, e: (0, 0)),
        ],
        out_shape=[
            jax.ShapeDtypeStruct((bl, 1), jnp.int32),
            jax.ShapeDtypeStruct((1, 1), jnp.float32),
        ],
        scratch_shapes=[
            pltpu.VMEM((TILE_T, 1), jnp.float32),
            pltpu.VMEM((TILE_T, 1), jnp.int32),
        ],
    )(z_flat, emb_t, xn, en)


def _sc_gather(embedding, idx):
    bl = idx.shape[0]
    d = embedding.shape[1]
    info = plsc.get_sparse_core_info()
    n_cores = info.num_cores
    n_workers = n_cores * info.num_subcores
    b_per_w = bl // n_workers
    chunk = 128
    n_chunks = b_per_w // chunk
    mesh = plsc.VectorSubcoreMesh(core_axis_name="c", subcore_axis_name="s")

    @functools.partial(
        pl.kernel,
        mesh=mesh,
        out_type=jax.ShapeDtypeStruct((bl, d), jnp.float32),
        scratch_types=[
            pltpu.VMEM((chunk,), jnp.int32),
            pltpu.VMEM((chunk, d), jnp.float32),
            pltpu.SemaphoreType.DMA,
        ],
    )
    def gather_kernel(emb_hbm, idx_hbm, out_hbm, idx_v, rows_v, sem):
        wid = lax.axis_index("s") * n_cores + lax.axis_index("c")
        base = wid * b_per_w
        for j in range(n_chunks):
            off = base + j * chunk
            pltpu.sync_copy(idx_hbm.at[pl.ds(off, chunk)], idx_v)
            pltpu.async_copy(emb_hbm.at[idx_v], rows_v, sem).wait()
            pltpu.sync_copy(rows_v, out_hbm.at[pl.ds(off, chunk)])

    return gather_kernel(embedding, idx)


def kernel(z, embedding):
    b, l, d = z.shape
    bl = b * l
    z_flat = z.reshape(bl, d)
    emb_t = embedding.T
    xn = jnp.sum(jnp.abs(z_flat) ** 2, axis=1, keepdims=True)
    en = jnp.sum(jnp.abs(embedding) ** 2, axis=1).reshape(1, -1)
    idx2, loss_sum = _distance_argmin(z_flat, emb_t, xn, en)
    idx = idx2.reshape(bl)
    emb_pad = jnp.pad(embedding, ((0, 0), (0, 128 - d)))
    zq = _sc_gather(emb_pad, idx)[:, :d]
    vq_loss = loss_sum[0, 0] * (2.0 / (bl * d))
    return zq.reshape(b, l, d), vq_loss


# speedup vs baseline: 1.3557x; 1.3557x over previous
import functools

import jax
import jax.numpy as jnp
from jax import lax
from jax.experimental import pallas as pl
from jax.experimental.pallas import tpu as pltpu
from jax.experimental.pallas import tpu_sc as plsc

TILE_T = 512
TILE_E = 4096


def _distance_argmin_kernel(z_ref, et_ref, xn_ref, en_ref, idx_ref, loss_ref,
                            best_ref, bidx_ref):
    e = pl.program_id(1)
    n_e = pl.num_programs(1)
    z = z_ref[...]
    et = et_ref[...]
    xn = xn_ref[...]
    en = en_ref[...]
    mm = lax.dot_general(z.astype(jnp.bfloat16), et, (((1,), (0,)), ((), ())),
                         preferred_element_type=jnp.float32)
    dist = xn - 2.0 * mm + en
    lmin = jnp.min(dist, axis=1, keepdims=True)
    ii = lax.broadcasted_iota(jnp.int32, dist.shape, 1)
    lidx = (jnp.min(jnp.where(dist == lmin, ii, dist.shape[1]),
                    axis=1, keepdims=True) + e * TILE_E)

    @pl.when(e == 0)
    def _():
        best_ref[...] = lmin
        bidx_ref[...] = lidx

    @pl.when(e > 0)
    def _():
        better = lmin < best_ref[...]
        best_ref[...] = jnp.where(better, lmin, best_ref[...])
        bidx_ref[...] = jnp.where(better, lidx, bidx_ref[...])

    @pl.when(e == n_e - 1)
    def _():
        idx_ref[...] = bidx_ref[...]
        part = jnp.sum(best_ref[...], axis=0, keepdims=True)
        t = pl.program_id(0)

        @pl.when(t == 0)
        def _():
            loss_ref[...] = part

        @pl.when(t > 0)
        def _():
            loss_ref[...] = loss_ref[...] + part


def _distance_argmin(z_flat, emb_t, xn, en):
    bl, d = z_flat.shape
    ne = emb_t.shape[1]
    grid = (bl // TILE_T, ne // TILE_E)
    return pl.pallas_call(
        _distance_argmin_kernel,
        grid=grid,
        in_specs=[
            pl.BlockSpec((TILE_T, d), lambda t, e: (t, 0)),
            pl.BlockSpec((d, TILE_E), lambda t, e: (0, e)),
            pl.BlockSpec((TILE_T, 1), lambda t, e: (t, 0)),
            pl.BlockSpec((1, TILE_E), lambda t, e: (0, e)),
        ],
        out_specs=[
            pl.BlockSpec((TILE_T, 1), lambda t, e: (t, 0)),
            pl.BlockSpec((1, 1), lambda t, e: (0, 0)),
        ],
        out_shape=[
            jax.ShapeDtypeStruct((bl, 1), jnp.int32),
            jax.ShapeDtypeStruct((1, 1), jnp.float32),
        ],
        scratch_shapes=[
            pltpu.VMEM((TILE_T, 1), jnp.float32),
            pltpu.VMEM((TILE_T, 1), jnp.int32),
        ],
    )(z_flat, emb_t, xn, en)


def _sc_gather(embedding, idx):
    bl = idx.shape[0]
    d = embedding.shape[1]
    info = plsc.get_sparse_core_info()
    n_cores = info.num_cores
    n_workers = n_cores * info.num_subcores
    b_per_w = bl // n_workers
    chunk = 128
    n_chunks = b_per_w // chunk
    mesh = plsc.VectorSubcoreMesh(core_axis_name="c", subcore_axis_name="s")

    @functools.partial(
        pl.kernel,
        mesh=mesh,
        out_type=jax.ShapeDtypeStruct((bl, d), jnp.float32),
        scratch_types=[
            pltpu.VMEM((chunk,), jnp.int32),
            pltpu.VMEM((chunk, d), jnp.float32),
            pltpu.SemaphoreType.DMA,
        ],
    )
    def gather_kernel(emb_hbm, idx_hbm, out_hbm, idx_v, rows_v, sem):
        wid = lax.axis_index("s") * n_cores + lax.axis_index("c")
        base = wid * b_per_w
        for j in range(n_chunks):
            off = base + j * chunk
            pltpu.sync_copy(idx_hbm.at[pl.ds(off, chunk)], idx_v)
            pltpu.async_copy(emb_hbm.at[idx_v], rows_v, sem).wait()
            pltpu.sync_copy(rows_v, out_hbm.at[pl.ds(off, chunk)])

    return gather_kernel(embedding, idx)


def kernel(z, embedding):
    b, l, d = z.shape
    bl = b * l
    z_flat = z.reshape(bl, d)
    emb_t = embedding.T
    xn = jnp.sum(jnp.abs(z_flat) ** 2, axis=1, keepdims=True)
    en = jnp.sum(jnp.abs(embedding) ** 2, axis=1).reshape(1, -1)
    idx2, loss_sum = _distance_argmin(z_flat, emb_t, xn, en)
    idx = idx2.reshape(bl)
    emb_pad = jnp.pad(embedding, ((0, 0), (0, 128 - d)))
    zq = _sc_gather(emb_pad, idx)[:, :d]
    vq_loss = loss_sum[0, 0] * (2.0 / (bl * d))
    return zq.reshape(b, l, d), vq_loss


# speedup vs baseline: 1.4149x; 1.0437x over previous
import functools

import jax
import jax.numpy as jnp
from jax import lax
from jax.experimental import pallas as pl
from jax.experimental.pallas import tpu as pltpu
from jax.experimental.pallas import tpu_sc as plsc

TILE_T = 1024
TILE_E = 4096


def _distance_argmin_kernel(z_ref, et_ref, xn_ref, en_ref, idx_ref, loss_ref,
                            best_ref, bidx_ref):
    e = pl.program_id(1)
    n_e = pl.num_programs(1)
    z = z_ref[...]
    et = et_ref[...]
    xn = xn_ref[...]
    en = en_ref[...]
    mm = lax.dot_general(z.astype(jnp.bfloat16), et, (((1,), (0,)), ((), ())),
                         preferred_element_type=jnp.float32)
    dist = xn - 2.0 * mm + en
    lmin = jnp.min(dist, axis=1, keepdims=True)
    ii = lax.broadcasted_iota(jnp.int32, dist.shape, 1)
    lidx = (jnp.min(jnp.where(dist == lmin, ii, dist.shape[1]),
                    axis=1, keepdims=True) + e * TILE_E)

    @pl.when(e == 0)
    def _():
        best_ref[...] = lmin
        bidx_ref[...] = lidx

    @pl.when(e > 0)
    def _():
        better = lmin < best_ref[...]
        best_ref[...] = jnp.where(better, lmin, best_ref[...])
        bidx_ref[...] = jnp.where(better, lidx, bidx_ref[...])

    @pl.when(e == n_e - 1)
    def _():
        idx_ref[...] = bidx_ref[...]
        part = jnp.sum(best_ref[...], axis=0, keepdims=True)
        t = pl.program_id(0)

        @pl.when(t == 0)
        def _():
            loss_ref[...] = part

        @pl.when(t > 0)
        def _():
            loss_ref[...] = loss_ref[...] + part


def _distance_argmin(z_flat, emb_t, xn, en):
    bl, d = z_flat.shape
    ne = emb_t.shape[1]
    grid = (bl // TILE_T, ne // TILE_E)
    return pl.pallas_call(
        _distance_argmin_kernel,
        grid=grid,
        in_specs=[
            pl.BlockSpec((TILE_T, d), lambda t, e: (t, 0)),
            pl.BlockSpec((d, TILE_E), lambda t, e: (0, e)),
            pl.BlockSpec((TILE_T, 1), lambda t, e: (t, 0)),
            pl.BlockSpec((1, TILE_E), lambda t, e: (0, e)),
        ],
        out_specs=[
            pl.BlockSpec((TILE_T, 1), lambda t, e: (t, 0)),
            pl.BlockSpec((1, 1), lambda t, e: (0, 0)),
        ],
        out_shape=[
            jax.ShapeDtypeStruct((bl, 1), jnp.int32),
            jax.ShapeDtypeStruct((1, 1), jnp.float32),
        ],
        scratch_shapes=[
            pltpu.VMEM((TILE_T, 1), jnp.float32),
            pltpu.VMEM((TILE_T, 1), jnp.int32),
        ],
    )(z_flat, emb_t, xn, en)


def _sc_gather(embedding, idx):
    bl = idx.shape[0]
    d = embedding.shape[1]
    info = plsc.get_sparse_core_info()
    n_cores = info.num_cores
    n_workers = n_cores * info.num_subcores
    b_per_w = bl // n_workers
    chunk = 128
    n_chunks = b_per_w // chunk
    mesh = plsc.VectorSubcoreMesh(core_axis_name="c", subcore_axis_name="s")

    @functools.partial(
        pl.kernel,
        mesh=mesh,
        out_type=jax.ShapeDtypeStruct((bl, d), jnp.float32),
        scratch_types=[
            pltpu.VMEM((chunk,), jnp.int32),
            pltpu.VMEM((chunk, d), jnp.float32),
            pltpu.SemaphoreType.DMA,
        ],
    )
    def gather_kernel(emb_hbm, idx_hbm, out_hbm, idx_v, rows_v, sem):
        wid = lax.axis_index("s") * n_cores + lax.axis_index("c")
        base = wid * b_per_w
        for j in range(n_chunks):
            off = base + j * chunk
            pltpu.sync_copy(idx_hbm.at[pl.ds(off, chunk)], idx_v)
            pltpu.async_copy(emb_hbm.at[idx_v], rows_v, sem).wait()
            pltpu.sync_copy(rows_v, out_hbm.at[pl.ds(off, chunk)])

    return gather_kernel(embedding, idx)


def kernel(z, embedding):
    b, l, d = z.shape
    bl = b * l
    z_flat = z.reshape(bl, d)
    emb_t = embedding.T
    xn = jnp.sum(jnp.abs(z_flat) ** 2, axis=1, keepdims=True)
    en = jnp.sum(jnp.abs(embedding) ** 2, axis=1).reshape(1, -1)
    idx2, loss_sum = _distance_argmin(z_flat, emb_t, xn, en)
    idx = idx2.reshape(bl)
    emb_pad = jnp.pad(embedding, ((0, 0), (0, 128 - d)))
    zq = _sc_gather(emb_pad, idx)[:, :d]
    vq_loss = loss_sum[0, 0] * (2.0 / (bl * d))
    return zq.reshape(b, l, d), vq_loss
